# baseline (speedup 1.0000x reference)
import functools
import math

import jax
import jax.numpy as jnp
from jax import lax
from jax.experimental import pallas as pl
from jax.experimental.pallas import tpu as pltpu
from jax.experimental.pallas import tpu_sc as plsc

B = 4096
L = 200
D = 64
SCALE = math.sqrt(D)

NW = 32
ROWS = B * L
PER_W = ROWS // NW
C = 128
G = PER_W // C
LANES = 16


def _sc_body(tok_hbm, table_hbm, out_hbm, idx_v, rows_v, sem):
    cid = lax.axis_index("c")
    sid = lax.axis_index("s")
    wid = sid * 2 + cid
    base = wid * PER_W

    pltpu.sync_copy(tok_hbm.at[wid], idx_v)

    def chunk(g, carry):
        pltpu.async_copy(table_hbm.at[idx_v.at[g]], rows_v, sem).wait()

        def scale_row(r, c2):
            for j in range(D // LANES):
                sl = pl.ds(j * LANES, LANES)
                rows_v[r, sl] = rows_v[r, sl] * jnp.float32(SCALE)
            return c2

        lax.fori_loop(0, C, scale_row, 0)

        pltpu.sync_copy(rows_v, out_hbm.at[pl.ds(base + g * C, C)])
        return carry

    lax.fori_loop(0, G, chunk, 0)


_sc_gather = functools.partial(
    pl.kernel,
    mesh=plsc.VectorSubcoreMesh(core_axis_name="c", subcore_axis_name="s"),
    out_type=jax.ShapeDtypeStruct((ROWS, D), jnp.float32),
    scratch_types=[
        pltpu.VMEM((G, C), jnp.int32),
        pltpu.VMEM((C, D), jnp.float32),
        pltpu.SemaphoreType.DMA,
    ],
    compiler_params=pltpu.CompilerParams(use_tc_tiling_on_sc=False),
)(_sc_body)


def kernel(tokens, table):
    tok = tokens.reshape(NW, G, C).astype(jnp.int32)
    out = _sc_gather(tok, table)
    return out.reshape(B, L, D)


# speedup vs baseline: 1.2064x; 1.2064x over previous
import functools
import math

import jax
import jax.numpy as jnp
from jax import lax
from jax.experimental import pallas as pl
from jax.experimental.pallas import tpu as pltpu
from jax.experimental.pallas import tpu_sc as plsc

B = 4096
L = 200
D = 64
SCALE = math.sqrt(D)

NW = 32
ROWS = B * L
PER_W = ROWS // NW
C = 128
G = PER_W // C
NBUF = 4
LANES = 16
RUNROLL = 8


def _sc_body(tok_hbm, table_hbm, out_hbm, idx_v, ins, outs, sem_g, sem_s):
    cid = lax.axis_index("c")
    sid = lax.axis_index("s")
    wid = sid * 2 + cid
    base = wid * PER_W

    pltpu.sync_copy(tok_hbm.at[wid], idx_v)

    def start_gather(b, g):
        pltpu.make_async_copy(
            table_hbm.at[idx_v.at[g]], ins[b], sem_g.at[b]
        ).start()

    def scale(b):
        src, dst = ins[b], outs[b]

        def rowblk(i, carry):
            r0 = i * RUNROLL
            for rr in range(RUNROLL):
                for j in range(D // LANES):
                    sl = pl.ds(j * LANES, LANES)
                    dst[r0 + rr, sl] = src[r0 + rr, sl] * jnp.float32(SCALE)
            return carry

        lax.fori_loop(0, C // RUNROLL, rowblk, 0)

    for b in range(NBUF):
        start_gather(b, b)

    def outer(t, carry):
        g0 = t * NBUF
        for b in range(NBUF):
            g = g0 + b
            pltpu.make_async_copy(
                table_hbm.at[idx_v.at[g]], ins[b], sem_g.at[b]
            ).wait()
            @pl.when(g >= NBUF)
            def _():
                pltpu.make_async_copy(
                    outs[b], out_hbm.at[pl.ds(base + (g - NBUF) * C, C)],
                    sem_s.at[b],
                ).wait()

            scale(b)

            @pl.when(g + NBUF < G)
            def _():
                start_gather(b, g + NBUF)

            pltpu.make_async_copy(
                outs[b], out_hbm.at[pl.ds(base + g * C, C)], sem_s.at[b]
            ).start()
        return carry

    lax.fori_loop(0, G // NBUF, outer, 0)

    for b in range(NBUF):
        g = G - NBUF + b
        pltpu.make_async_copy(
            outs[b], out_hbm.at[pl.ds(base + g * C, C)], sem_s.at[b]
        ).wait()


_sc_gather = functools.partial(
    pl.kernel,
    mesh=plsc.VectorSubcoreMesh(core_axis_name="c", subcore_axis_name="s"),
    out_type=jax.ShapeDtypeStruct((ROWS, D), jnp.float32),
    scratch_types=[
        pltpu.VMEM((G, C), jnp.int32),
        [pltpu.VMEM((C, D), jnp.float32) for _ in range(NBUF)],
        [pltpu.VMEM((C, D), jnp.float32) for _ in range(NBUF)],
        pltpu.SemaphoreType.DMA((NBUF,)),
        pltpu.SemaphoreType.DMA((NBUF,)),
    ],
    compiler_params=pltpu.CompilerParams(use_tc_tiling_on_sc=False),
)(_sc_body)


def kernel(tokens, table):
    tok = tokens.reshape(NW, G, C).astype(jnp.int32)
    out = _sc_gather(tok, table)
    return out.reshape(B, L, D)
